# baseline (speedup 1.0000x reference)
import numpy as np
import jax
import jax.numpy as jnp
from jax.experimental import pallas as pl
from jax.experimental.pallas import tpu as pltpu

REG_MAX = 16
NM = 32
STRIDES = (8.0, 16.0, 32.0)
LEVELS = ((80, 80), (40, 40), (20, 20))
TOPK = 10
BETA = 6.0
BOX_W, CLS_W, MASK_W, DFL_W = 7.5, 0.5, 2.5, 1.5
EPS = 1e-9
NA = sum(h * w for h, w in LEVELS)
NGT = 12
NPIX = 160 * 160
PIX_CHUNK = 6400


def _make_anchor_rows():
    ax, ay, st = [], [], []
    for (h, w), s in zip(LEVELS, STRIDES):
        xs = (np.arange(w, dtype=np.float32) + 0.5) * s
        ys = (np.arange(h, dtype=np.float32) + 0.5) * s
        ax.append(np.tile(xs, h))
        ay.append(np.repeat(ys, w))
        st.append(np.full(h * w, s, dtype=np.float32))
    rows = np.zeros((8, NA), dtype=np.float32)
    rows[0] = np.concatenate(ax)
    rows[1] = np.concatenate(ay)
    rows[2] = np.concatenate(st)
    return jnp.asarray(rows)


def _bce(x, z):
    return jnp.maximum(x, 0.0) - x * z + jnp.log1p(jnp.exp(-jnp.abs(x)))


def _loss_kernel(anch_ref, gtb_ref, box_ref, cls_ref, mc_ref, proto_ref,
                 gtm_ref, out_ref):
    ax = anch_ref[0:1, :]
    ay = anch_ref[1:2, :]
    stv = anch_ref[2:3, :]

    proj = jax.lax.broadcasted_iota(jnp.int32, (REG_MAX, 1), 0).astype(
        jnp.float32)
    dists = []
    logps = []
    for k in range(4):
        lg = box_ref[0, REG_MAX * k:REG_MAX * (k + 1), :]
        mx = jnp.max(lg, axis=0, keepdims=True)
        ex = jnp.exp(lg - mx)
        s = jnp.sum(ex, axis=0, keepdims=True)
        dists.append(jnp.sum((ex / s) * proj, axis=0, keepdims=True))
        logps.append((lg - mx) - jnp.log(s))
    d_l = dists[0] * stv
    d_t = dists[1] * stv
    d_r = dists[2] * stv
    d_b = dists[3] * stv
    x1 = ax - d_l
    y1 = ay - d_t
    x2 = ax + d_r
    y2 = ay + d_b

    score = jax.nn.sigmoid(cls_ref[0])

    g_x1 = gtb_ref[0, :, 0:1]
    g_y1 = gtb_ref[0, :, 1:2]
    g_x2 = gtb_ref[0, :, 2:3]
    g_y2 = gtb_ref[0, :, 3:4]

    iw = jnp.clip(jnp.minimum(g_x2, x2) - jnp.maximum(g_x1, x1), 0.0, None)
    ih = jnp.clip(jnp.minimum(g_y2, y2) - jnp.maximum(g_y1, y1), 0.0, None)
    inter = iw * ih
    area_g = (g_x2 - g_x1) * (g_y2 - g_y1)
    area_d = (x2 - x1) * (y2 - y1)
    ious = inter / (area_g + area_d - inter + EPS)

    i2 = ious * ious
    align = score * (i2 * i2 * i2)
    in_gt = (ax > g_x1) & (ax < g_x2) & (ay > g_y1) & (ay < g_y2)
    metric = jnp.where(in_gt, align, 0.0)

    lane_iota = jax.lax.broadcasted_iota(jnp.int32, (NGT, NA), 1)
    sel_rows = []
    cur = metric
    for _ in range(TOPK):
        mx = jnp.max(cur, axis=1, keepdims=True)
        pos = jnp.where(cur == mx, lane_iota, NA)
        mp = jnp.min(pos, axis=1, keepdims=True)
        first = lane_iota == mp
        sel_rows.append(jnp.where(first & (mx > 0.0), 1.0, 0.0))
        cur = jnp.where(first, -1.0, cur)

    mask_pos = sel_rows[0]
    for k in range(1, TOPK):
        mask_pos = mask_pos + sel_rows[k]
    fgf = jnp.minimum(jnp.sum(mask_pos, axis=0, keepdims=True), 1.0)

    iou_m = jnp.where(mask_pos > 0.0, ious, -1.0)
    best = iou_m[0:1, :]
    bidx = jnp.zeros((1, NA), jnp.int32)
    for g in range(1, NGT):
        v = iou_m[g:g + 1, :]
        take = v > best
        best = jnp.where(take, v, best)
        bidx = jnp.where(take, g, bidx)
    sub_iota = jax.lax.broadcasted_iota(jnp.int32, (NGT, NA), 0)
    oh = jnp.where(sub_iota == bidx, 1.0, 0.0)

    bg_x1 = jnp.sum(oh * g_x1, axis=0, keepdims=True)
    bg_y1 = jnp.sum(oh * g_y1, axis=0, keepdims=True)
    bg_x2 = jnp.sum(oh * g_x2, axis=0, keepdims=True)
    bg_y2 = jnp.sum(oh * g_y2, axis=0, keepdims=True)

    eiw = jnp.clip(jnp.minimum(bg_x2, x2) - jnp.maximum(bg_x1, x1), 0.0, None)
    eih = jnp.clip(jnp.minimum(bg_y2, y2) - jnp.maximum(bg_y1, y1), 0.0, None)
    einter = eiw * eih
    area_b = (bg_x2 - bg_x1) * (bg_y2 - bg_y1)
    iou_e = einter / (area_d + area_b - einter + EPS)
    l_box = jnp.sum((1.0 - iou_e) * fgf)

    tgt = jnp.clip(iou_e, 0.0, 1.0)
    l_cls = jnp.sum(_bce(cls_ref[0], tgt) * fgf)

    l_dfl = 0.0
    tvals = ((ax - bg_x1) / stv, (ay - bg_y1) / stv,
             (bg_x2 - ax) / stv, (bg_y2 - ay) / stv)
    bin_iota = jax.lax.broadcasted_iota(jnp.int32, (REG_MAX, NA), 0)
    for k in range(4):
        t = jnp.clip(tvals[k], 0.0, REG_MAX - 1e-6)
        tl = t.astype(jnp.int32)
        tr = jnp.minimum(tl + 1, REG_MAX - 1)
        at_b = tl >= REG_MAX - 1
        tr = jnp.where(at_b, tl, tr)
        wr = t - tl.astype(jnp.float32)
        wl = 1.0 - wr
        wr = jnp.where(at_b, 0.0, wr)
        wl = jnp.where(at_b, 1.0, wl)
        logp = logps[k]
        ce_l = -jnp.sum(jnp.where(bin_iota == tl, logp, 0.0), axis=0,
                        keepdims=True)
        ce_r = -jnp.sum(jnp.where(bin_iota == tr, logp, 0.0), axis=0,
                        keepdims=True)
        l_dfl = l_dfl + jnp.sum((ce_l * wl + ce_r * wr) * fgf)

    sel120 = jnp.concatenate(sel_rows, axis=0)
    wsel = jnp.concatenate(
        [jnp.sum(sel_rows[k] * oh, axis=1, keepdims=True) for k in range(TOPK)],
        axis=0)
    mc_sel = jax.lax.dot_general(sel120, mc_ref[0],
                                 (((1,), (1,)), ((), ())),
                                 preferred_element_type=jnp.float32)
    eye12 = jnp.where(
        jax.lax.broadcasted_iota(jnp.int32, (NGT, NGT), 0)
        == jax.lax.broadcasted_iota(jnp.int32, (NGT, NGT), 1), 1.0, 0.0)
    tsel = jnp.concatenate([eye12] * TOPK, axis=0)
    agg = jax.lax.dot_general(tsel, mc_sel * wsel,
                              (((0,), (0,)), ((), ())),
                              preferred_element_type=jnp.float32)
    acc = 0.0
    cross = jnp.zeros((NGT, NM), jnp.float32)
    for c in range(NPIX // PIX_CHUNK):
        pchunk = proto_ref[0, :, PIX_CHUNK * c:PIX_CHUNK * (c + 1)]
        gchunk = gtm_ref[0, :, PIX_CHUNK * c:PIX_CHUNK * (c + 1)]
        pm = jnp.dot(mc_sel, pchunk, preferred_element_type=jnp.float32)
        sp = jnp.maximum(pm, 0.0) + jnp.log1p(jnp.exp(-jnp.abs(pm)))
        acc = acc + jnp.sum(sp * wsel)
        cross = cross + jax.lax.dot_general(
            gchunk, pchunk, (((1,), (1,)), ((), ())),
            preferred_element_type=jnp.float32)
    l_msk = (acc - jnp.sum(agg * cross)) / float(NPIX)

    num_pos = jnp.sum(fgf)

    oiota = jax.lax.broadcasted_iota(jnp.int32, (1, 128), 1)
    vec = (jnp.where(oiota == 0, l_box, 0.0)
           + jnp.where(oiota == 1, l_cls, 0.0)
           + jnp.where(oiota == 2, l_dfl, 0.0)
           + jnp.where(oiota == 3, l_msk, 0.0)
           + jnp.where(oiota == 4, num_pos, 0.0))
    out_ref[0, :, :] = vec


def kernel(box_p3, box_p4, box_p5, cls_p3, cls_p4, cls_p5, mc_p3, mc_p4,
           mc_p5, proto, gt_boxes, gt_masks):
    B = box_p3.shape[0]
    box_flat = jnp.concatenate(
        [p.reshape(B, 4 * REG_MAX, -1) for p in (box_p3, box_p4, box_p5)],
        axis=-1)
    cls_flat = jnp.concatenate(
        [p.reshape(B, 1, -1) for p in (cls_p3, cls_p4, cls_p5)], axis=-1)
    mc_flat = jnp.concatenate(
        [p.reshape(B, NM, -1) for p in (mc_p3, mc_p4, mc_p5)], axis=-1)
    proto_r = proto.reshape(B, NM, NPIX)
    gtm_r = gt_masks.reshape(B, NGT, NPIX)
    anch = _make_anchor_rows()

    out = pl.pallas_call(
        _loss_kernel,
        grid=(B,),
        in_specs=[
            pl.BlockSpec((8, NA), lambda i: (0, 0)),
            pl.BlockSpec((1, NGT, 4), lambda i: (i, 0, 0)),
            pl.BlockSpec((1, 4 * REG_MAX, NA), lambda i: (i, 0, 0)),
            pl.BlockSpec((1, 1, NA), lambda i: (i, 0, 0)),
            pl.BlockSpec((1, NM, NA), lambda i: (i, 0, 0)),
            pl.BlockSpec((1, NM, NPIX), lambda i: (i, 0, 0)),
            pl.BlockSpec((1, NGT, NPIX), lambda i: (i, 0, 0)),
        ],
        out_specs=pl.BlockSpec((1, 1, 128), lambda i: (i, 0, 0)),
        out_shape=jax.ShapeDtypeStruct((B, 1, 128), jnp.float32),
        compiler_params=pltpu.CompilerParams(
            dimension_semantics=("parallel",)),
    )(anch, gt_boxes, box_flat, cls_flat, mc_flat, proto_r, gtm_r)

    l_box = jnp.sum(out[:, 0, 0])
    l_cls = jnp.sum(out[:, 0, 1])
    l_dfl = jnp.sum(out[:, 0, 2])
    l_msk = jnp.sum(out[:, 0, 3])
    num_pos = jnp.sum(out[:, 0, 4])
    return (BOX_W * l_box / num_pos + CLS_W * l_cls / num_pos
            + MASK_W * l_msk / num_pos + DFL_W * l_dfl / (num_pos * 4.0))


# speedup vs baseline: 1.0939x; 1.0939x over previous
import numpy as np
import jax
import jax.numpy as jnp
from jax.experimental import pallas as pl
from jax.experimental.pallas import tpu as pltpu

REG_MAX = 16
NM = 32
STRIDES = (8.0, 16.0, 32.0)
LEVELS = ((80, 80), (40, 40), (20, 20))
TOPK = 10
BOX_W, CLS_W, MASK_W, DFL_W = 7.5, 0.5, 2.5, 1.5
EPS = 1e-9
NA = sum(h * w for h, w in LEVELS)
NGT = 12
NPIX = 160 * 160
PIX_CHUNK = 6400
LEVEL_NA = tuple(h * w for h, w in LEVELS)
LEVEL_OFF = (0, LEVEL_NA[0], LEVEL_NA[0] + LEVEL_NA[1])


def _make_anchor_rows(level):
    (h, w), s = LEVELS[level], STRIDES[level]
    rows = np.zeros((8, h * w), dtype=np.float32)
    rows[0] = np.tile((np.arange(w, dtype=np.float32) + 0.5) * s, h)
    rows[1] = np.repeat((np.arange(h, dtype=np.float32) + 0.5) * s, w)
    rows[2] = np.full(h * w, s, dtype=np.float32)
    return jnp.asarray(rows)


def _loss_kernel(a3, a4, a5, gtb_ref, b3, b4, b5, c3, c4, c5, m3, m4, m5,
                 proto_ref, gtm_ref, out_ref):
    g_x1 = gtb_ref[0, :, 0:1]
    g_y1 = gtb_ref[0, :, 1:2]
    g_x2 = gtb_ref[0, :, 2:3]
    g_y2 = gtb_ref[0, :, 3:4]
    area_g = (g_x2 - g_x1) * (g_y2 - g_y1)

    proj = jax.lax.broadcasted_iota(jnp.int32, (REG_MAX, 1), 0).astype(
        jnp.float32)

    lv = []
    for aref, bref, cref in ((a3, b3, c3), (a4, b4, c4), (a5, b5, c5)):
        ax = aref[0:1, :]
        ay = aref[1:2, :]
        stv = aref[2:3, :]
        na = ax.shape[1]

        dists = []
        logps = []
        for k in range(4):
            lg = bref[0, REG_MAX * k:REG_MAX * (k + 1), :]
            mx = jnp.max(lg, axis=0, keepdims=True)
            ex = jnp.exp(lg - mx)
            s = jnp.sum(ex, axis=0, keepdims=True)
            dists.append(jnp.sum((ex / s) * proj, axis=0, keepdims=True))
            logps.append((lg - mx) - jnp.log(s))
        x1 = ax - dists[0] * stv
        y1 = ay - dists[1] * stv
        x2 = ax + dists[2] * stv
        y2 = ay + dists[3] * stv

        score = jax.nn.sigmoid(cref[0])

        iw = jnp.clip(jnp.minimum(g_x2, x2) - jnp.maximum(g_x1, x1), 0.0,
                      None)
        ih = jnp.clip(jnp.minimum(g_y2, y2) - jnp.maximum(g_y1, y1), 0.0,
                      None)
        inter = iw * ih
        area_d = (x2 - x1) * (y2 - y1)
        ious = inter / (area_g + area_d - inter + EPS)

        i2 = ious * ious
        align = score * (i2 * i2 * i2)
        in_gt = ((ax > g_x1) & (ax < g_x2) & (ay > g_y1) & (ay < g_y2))
        metric = jnp.where(in_gt, align, 0.0)
        lane_iota = jax.lax.broadcasted_iota(jnp.int32, (NGT, na), 1)
        lv.append(dict(ax=ax, ay=ay, stv=stv, logps=logps, x1=x1, y1=y1,
                       x2=x2, y2=y2, area_d=area_d, ious=ious,
                       metric=metric, iota=lane_iota, na=na))

    sels = [[], [], []]
    curs = [l['metric'] for l in lv]
    for _ in range(TOPK):
        mxs = [jnp.max(c, axis=1, keepdims=True) for c in curs]
        mx = jnp.maximum(jnp.maximum(mxs[0], mxs[1]), mxs[2])
        mps = [jnp.min(jnp.where(curs[i] == mx, lv[i]['iota'] + LEVEL_OFF[i],
                                 NA), axis=1, keepdims=True)
               for i in range(3)]
        mp = jnp.minimum(jnp.minimum(mps[0], mps[1]), mps[2])
        pos_ok = mx > 0.0
        for i in range(3):
            first = (lv[i]['iota'] + LEVEL_OFF[i]) == mp
            sels[i].append(jnp.where(first & pos_ok, 1.0, 0.0))
            curs[i] = jnp.where(first, -1.0, curs[i])

    l_box = 0.0
    l_cls = 0.0
    l_dfl = 0.0
    num_pos = 0.0
    wsel_parts = [None, None, None]
    for i in range(3):
        l = lv[i]
        na = l['na']
        mask_pos = sels[i][0]
        for k in range(1, TOPK):
            mask_pos = mask_pos + sels[i][k]
        fgf = jnp.minimum(jnp.sum(mask_pos, axis=0, keepdims=True), 1.0)
        num_pos = num_pos + jnp.sum(fgf)

        iou_m = jnp.where(mask_pos > 0.0, l['ious'], -1.0)
        best = iou_m[0:1, :]
        bidx = jnp.zeros((1, na), jnp.int32)
        for g in range(1, NGT):
            v = iou_m[g:g + 1, :]
            take = v > best
            best = jnp.where(take, v, best)
            bidx = jnp.where(take, g, bidx)
        sub_iota = jax.lax.broadcasted_iota(jnp.int32, (NGT, na), 0)
        oh = jnp.where(sub_iota == bidx, 1.0, 0.0)

        bg_x1 = jnp.sum(oh * g_x1, axis=0, keepdims=True)
        bg_y1 = jnp.sum(oh * g_y1, axis=0, keepdims=True)
        bg_x2 = jnp.sum(oh * g_x2, axis=0, keepdims=True)
        bg_y2 = jnp.sum(oh * g_y2, axis=0, keepdims=True)

        eiw = jnp.clip(jnp.minimum(bg_x2, l['x2'])
                       - jnp.maximum(bg_x1, l['x1']), 0.0, None)
        eih = jnp.clip(jnp.minimum(bg_y2, l['y2'])
                       - jnp.maximum(bg_y1, l['y1']), 0.0, None)
        einter = eiw * eih
        area_b = (bg_x2 - bg_x1) * (bg_y2 - bg_y1)
        iou_e = einter / (l['area_d'] + area_b - einter + EPS)
        l_box = l_box + jnp.sum((1.0 - iou_e) * fgf)

        clogit_ref = (c3, c4, c5)[i]
        x = clogit_ref[0]
        tgt = jnp.clip(iou_e, 0.0, 1.0)
        bce = (jnp.maximum(x, 0.0) - x * tgt
               + jnp.log1p(jnp.exp(-jnp.abs(x))))
        l_cls = l_cls + jnp.sum(bce * fgf)

        tvals = ((l['ax'] - bg_x1) / l['stv'], (l['ay'] - bg_y1) / l['stv'],
                 (bg_x2 - l['ax']) / l['stv'], (bg_y2 - l['ay']) / l['stv'])
        bin_iota = jax.lax.broadcasted_iota(jnp.int32, (REG_MAX, na), 0)
        for k in range(4):
            t = jnp.clip(tvals[k], 0.0, REG_MAX - 1e-6)
            tl = t.astype(jnp.int32)
            tr = jnp.minimum(tl + 1, REG_MAX - 1)
            at_b = tl >= REG_MAX - 1
            tr = jnp.where(at_b, tl, tr)
            wr = t - tl.astype(jnp.float32)
            wl = 1.0 - wr
            wr = jnp.where(at_b, 0.0, wr)
            wl = jnp.where(at_b, 1.0, wl)
            logp = l['logps'][k]
            ce_l = -jnp.sum(jnp.where(bin_iota == tl, logp, 0.0), axis=0,
                            keepdims=True)
            ce_r = -jnp.sum(jnp.where(bin_iota == tr, logp, 0.0), axis=0,
                            keepdims=True)
            l_dfl = l_dfl + jnp.sum((ce_l * wl + ce_r * wr) * fgf)

        wsel_parts[i] = jnp.concatenate(
            [jnp.sum(sels[i][k] * oh, axis=1, keepdims=True)
             for k in range(TOPK)], axis=0)

    wsel = wsel_parts[0] + wsel_parts[1] + wsel_parts[2]
    mc_sel = jnp.zeros((TOPK * NGT, NM), jnp.float32)
    for i, mref in enumerate((m3, m4, m5)):
        sel120 = jnp.concatenate(sels[i], axis=0)
        mc_sel = mc_sel + jax.lax.dot_general(
            sel120, mref[0], (((1,), (1,)), ((), ())),
            preferred_element_type=jnp.float32)

    eye12 = jnp.where(
        jax.lax.broadcasted_iota(jnp.int32, (NGT, NGT), 0)
        == jax.lax.broadcasted_iota(jnp.int32, (NGT, NGT), 1), 1.0, 0.0)
    tsel = jnp.concatenate([eye12] * TOPK, axis=0)
    agg = jax.lax.dot_general(tsel, mc_sel * wsel,
                              (((0,), (0,)), ((), ())),
                              preferred_element_type=jnp.float32)
    acc = 0.0
    cross = jnp.zeros((NGT, NM), jnp.float32)
    for c in range(NPIX // PIX_CHUNK):
        pchunk = proto_ref[0, :, PIX_CHUNK * c:PIX_CHUNK * (c + 1)]
        gchunk = gtm_ref[0, :, PIX_CHUNK * c:PIX_CHUNK * (c + 1)]
        pm = jnp.dot(mc_sel, pchunk, preferred_element_type=jnp.float32)
        sp = jnp.maximum(pm, 0.0) + jnp.log1p(jnp.exp(-jnp.abs(pm)))
        acc = acc + jnp.sum(sp * wsel)
        cross = cross + jax.lax.dot_general(
            gchunk, pchunk, (((1,), (1,)), ((), ())),
            preferred_element_type=jnp.float32)
    l_msk = (acc - jnp.sum(agg * cross)) / float(NPIX)

    oiota = jax.lax.broadcasted_iota(jnp.int32, (1, 128), 1)
    vec = (jnp.where(oiota == 0, l_box, 0.0)
           + jnp.where(oiota == 1, l_cls, 0.0)
           + jnp.where(oiota == 2, l_dfl, 0.0)
           + jnp.where(oiota == 3, l_msk, 0.0)
           + jnp.where(oiota == 4, num_pos, 0.0))
    out_ref[0, :, :] = vec


def kernel(box_p3, box_p4, box_p5, cls_p3, cls_p4, cls_p5, mc_p3, mc_p4,
           mc_p5, proto, gt_boxes, gt_masks):
    B = box_p3.shape[0]
    boxes = [p.reshape(B, 4 * REG_MAX, -1) for p in (box_p3, box_p4, box_p5)]
    clss = [p.reshape(B, 1, -1) for p in (cls_p3, cls_p4, cls_p5)]
    mcs = [p.reshape(B, NM, -1) for p in (mc_p3, mc_p4, mc_p5)]
    proto_r = proto.reshape(B, NM, NPIX)
    gtm_r = gt_masks.reshape(B, NGT, NPIX)
    anchs = [_make_anchor_rows(i) for i in range(3)]

    in_specs = (
        [pl.BlockSpec((8, LEVEL_NA[i]), lambda b: (0, 0)) for i in range(3)]
        + [pl.BlockSpec((1, NGT, 4), lambda b: (b, 0, 0))]
        + [pl.BlockSpec((1, 4 * REG_MAX, LEVEL_NA[i]), lambda b: (b, 0, 0))
           for i in range(3)]
        + [pl.BlockSpec((1, 1, LEVEL_NA[i]), lambda b: (b, 0, 0))
           for i in range(3)]
        + [pl.BlockSpec((1, NM, LEVEL_NA[i]), lambda b: (b, 0, 0))
           for i in range(3)]
        + [pl.BlockSpec((1, NM, NPIX), lambda b: (b, 0, 0)),
           pl.BlockSpec((1, NGT, NPIX), lambda b: (b, 0, 0))]
    )

    out = pl.pallas_call(
        _loss_kernel,
        grid=(B,),
        in_specs=in_specs,
        out_specs=pl.BlockSpec((1, 1, 128), lambda b: (b, 0, 0)),
        out_shape=jax.ShapeDtypeStruct((B, 1, 128), jnp.float32),
        compiler_params=pltpu.CompilerParams(
            dimension_semantics=("parallel",)),
    )(anchs[0], anchs[1], anchs[2], gt_boxes, boxes[0], boxes[1], boxes[2],
      clss[0], clss[1], clss[2], mcs[0], mcs[1], mcs[2], proto_r, gtm_r)

    l_box = jnp.sum(out[:, 0, 0])
    l_cls = jnp.sum(out[:, 0, 1])
    l_dfl = jnp.sum(out[:, 0, 2])
    l_msk = jnp.sum(out[:, 0, 3])
    num_pos = jnp.sum(out[:, 0, 4])
    return (BOX_W * l_box / num_pos + CLS_W * l_cls / num_pos
            + MASK_W * l_msk / num_pos + DFL_W * l_dfl / (num_pos * 4.0))
